# baseline (speedup 1.0000x reference)
import functools

import jax
import jax.numpy as jnp
from jax import lax
from jax.experimental import pallas as pl
from jax.experimental.pallas import tpu as pltpu
from jax.experimental.pallas import tpu_sc as plsc

D_GRP = 8
N_CODES = 512
M_BLK = 512


def _pq_onehot(g, cbt, c2, rows):
    s = jax.lax.dot_general(
        g, cbt, (((1,), (0,)), ((), ())),
        preferred_element_type=jnp.float32,
    )
    g2 = jnp.sum(g * g, axis=1, keepdims=True)
    d2 = g2 - 2.0 * s + c2
    m = jnp.min(d2, axis=1, keepdims=True)
    ii = lax.broadcasted_iota(jnp.int32, (rows, N_CODES), 1)
    sel = jnp.min(jnp.where(d2 == m, ii, N_CODES), axis=1, keepdims=True)
    return (ii == sel).astype(jnp.float32)


def _dequant(oh, cb):
    return jax.lax.dot_general(
        oh, cb, (((1,), (0,)), ((), ())),
        preferred_element_type=jnp.float32,
    )


def _sc_gather(table, idx):
    n_chunks, chunk = idx.shape
    n_ids = n_chunks * chunk
    d = table.shape[1]
    info = plsc.get_sparse_core_info()
    nw = info.num_cores * info.num_subcores
    c_per_w = n_chunks // nw
    b_per_w = c_per_w * chunk
    mesh = plsc.VectorSubcoreMesh(core_axis_name="c", subcore_axis_name="s")

    @functools.partial(
        pl.kernel,
        mesh=mesh,
        out_type=jax.ShapeDtypeStruct((n_ids, d), jnp.float32),
        scratch_types=[
            pltpu.VMEM((c_per_w, chunk), jnp.int32),
            pltpu.VMEM((b_per_w, d), jnp.float32),
            pltpu.SemaphoreType.DMA,
        ],
    )
    def k(table_hbm, idx_hbm, out_hbm, idx_v, rows_v, sem):
        wid = lax.axis_index("s") * info.num_cores + lax.axis_index("c")
        pltpu.sync_copy(idx_hbm.at[pl.ds(wid * c_per_w, c_per_w)], idx_v)
        copies = [
            pltpu.async_copy(
                table_hbm.at[idx_v.at[j]],
                rows_v.at[pl.ds(j * chunk, chunk)], sem)
            for j in range(c_per_w)
        ]
        for c in copies:
            c.wait()
        pltpu.sync_copy(rows_v, out_hbm.at[pl.ds(wid * b_per_w, b_per_w)])

    return k(table, idx)


def _quantize_b_body(b_ref, rs_ref, cbt_ref, cb_ref, c2_ref, out_ref):
    g_all = b_ref[...] / rs_ref[...]
    cbt, cb, c2 = cbt_ref[...], cb_ref[...], c2_ref[...]
    rows = g_all.shape[0]
    deqs = []
    for j in range(128 // D_GRP):
        g = g_all[:, D_GRP * j:D_GRP * (j + 1)]
        oh = _pq_onehot(g, cbt, c2, rows)
        deqs.append(_dequant(oh, cb))
    out_ref[...] = jnp.concatenate(deqs, axis=1) * rs_ref[...]


def _quantize_b(b2d, rs_rep, cbt, cb, c2):
    n, w = b2d.shape
    return pl.pallas_call(
        _quantize_b_body,
        out_shape=jax.ShapeDtypeStruct((n, w), jnp.float32),
    )(b2d, rs_rep, cbt, cb, c2)


def _fused_body(i_u, a_ref, bq_ref, cbt_ref, cb_ref, c2_ref, out_ref):
    x = a_ref[...]
    u = x[:, :i_u]
    rs = x[:, i_u:i_u + 1]
    g_all = u / rs
    cbt, cb, c2 = cbt_ref[...], cb_ref[...], c2_ref[...]
    deqs = []
    for j in range(i_u // D_GRP):
        g = g_all[:, D_GRP * j:D_GRP * (j + 1)]
        oh = _pq_onehot(g, cbt, c2, M_BLK)
        deqs.append(_dequant(oh, cb))
    uq = jnp.concatenate(deqs, axis=1) * rs
    out_ref[...] = jax.lax.dot_general(
        uq, bq_ref[...], (((1,), (0,)), ((), ())),
        preferred_element_type=jnp.float32,
    )


def _fused_matmul(a, bq, cbt, cb, c2):
    m = a.shape[0]
    k, n = bq.shape
    return pl.pallas_call(
        functools.partial(_fused_body, k),
        grid=(m // M_BLK,),
        in_specs=[
            pl.BlockSpec((M_BLK, a.shape[1]), lambda i: (i, 0)),
            pl.BlockSpec((k, n), lambda i: (0, 0)),
            pl.BlockSpec(cbt.shape, lambda i: (0, 0)),
            pl.BlockSpec(cb.shape, lambda i: (0, 0)),
            pl.BlockSpec(c2.shape, lambda i: (0, 0)),
        ],
        out_specs=pl.BlockSpec((M_BLK, n), lambda i: (i, 0)),
        out_shape=jax.ShapeDtypeStruct((m, n), jnp.float32),
        compiler_params=pltpu.CompilerParams(
            dimension_semantics=("parallel",)),
    )(a, bq, cbt, cb, c2)


def kernel(local_ids, U, B, rs_U, rs_B, codebook):
    o_u, i_u = U.shape
    o_b, i_b = B.shape
    cbt = codebook.T
    c2 = (codebook * codebook).sum(-1)[None, :]

    table = jnp.pad(jnp.concatenate([U, rs_U], axis=1),
                    ((0, 0), (0, 128 - i_u - 1)))
    ids2d = local_ids.astype(jnp.int32).reshape(-1, 128)
    ug_raw = _sc_gather(table, ids2d)

    b2d = B.reshape(-1, 128)
    rs_rep = jnp.repeat(rs_B, i_b // 128, axis=0)
    bq = _quantize_b(b2d, rs_rep, cbt, cb=codebook, c2=c2).reshape(o_b, i_b)

    return _fused_matmul(ug_raw, bq, cbt, codebook, c2)


# speedup vs baseline: 1.5891x; 1.5891x over previous
import functools

import jax
import jax.numpy as jnp
from jax import lax
from jax.experimental import pallas as pl
from jax.experimental.pallas import tpu as pltpu
from jax.experimental.pallas import tpu_sc as plsc

D_GRP = 8
N_CODES = 512
M_BLK = 512
QU_BLK = 1024


def _pq_onehot(g, cbt_m2, c2, rows):
    s = jax.lax.dot_general(
        g, cbt_m2, (((1,), (0,)), ((), ())),
        preferred_element_type=jnp.float32,
    )
    g2 = jnp.sum(g * g, axis=1, keepdims=True)
    d2 = g2 + s + c2
    m = jnp.min(d2, axis=1, keepdims=True)
    ii = lax.broadcasted_iota(jnp.int32, (rows, N_CODES), 1)
    sel = jnp.min(jnp.where(d2 == m, ii, N_CODES), axis=1, keepdims=True)
    return (ii == sel).astype(jnp.float32)


def _dequant(oh, cb):
    return jax.lax.dot_general(
        oh, cb, (((1,), (0,)), ((), ())),
        preferred_element_type=jnp.float32,
    )


def _quantize_u_body(u_ref, rs_ref, cbt_ref, cb_ref, c2_ref, out_ref):
    rs = rs_ref[...]
    g_all = u_ref[...] / rs
    cbt, cb, c2 = cbt_ref[...], cb_ref[...], c2_ref[...]
    deqs = []
    for j in range(32 // D_GRP):
        g = g_all[:, D_GRP * j:D_GRP * (j + 1)]
        oh = _pq_onehot(g, cbt, c2, QU_BLK)
        deqs.append(_dequant(oh, cb))
    uq = jnp.concatenate(deqs, axis=1) * rs
    out_ref[...] = jnp.pad(uq, ((0, 0), (0, 96)))


def _quantize_u(u, rs_u, cbt_m2, cb, c2):
    n = u.shape[0]
    return pl.pallas_call(
        _quantize_u_body,
        grid=(n // QU_BLK,),
        in_specs=[
            pl.BlockSpec((QU_BLK, 32), lambda i: (i, 0)),
            pl.BlockSpec((QU_BLK, 1), lambda i: (i, 0)),
            pl.BlockSpec(cbt_m2.shape, lambda i: (0, 0)),
            pl.BlockSpec(cb.shape, lambda i: (0, 0)),
            pl.BlockSpec(c2.shape, lambda i: (0, 0)),
        ],
        out_specs=pl.BlockSpec((QU_BLK, 128), lambda i: (i, 0)),
        out_shape=jax.ShapeDtypeStruct((n, 128), jnp.float32),
    )(u, rs_u, cbt_m2, cb, c2)


def _sc_gather(table, idx):
    n_chunks, chunk = idx.shape
    n_ids = n_chunks * chunk
    d = table.shape[1]
    info = plsc.get_sparse_core_info()
    nw = info.num_cores * info.num_subcores
    c_per_w = n_chunks // nw
    b_per_w = c_per_w * chunk
    mesh = plsc.VectorSubcoreMesh(core_axis_name="c", subcore_axis_name="s")

    @functools.partial(
        pl.kernel,
        mesh=mesh,
        out_type=jax.ShapeDtypeStruct((n_ids, d), jnp.float32),
        scratch_types=[
            pltpu.VMEM((c_per_w, chunk), jnp.int32),
            pltpu.VMEM((b_per_w, d), jnp.float32),
            pltpu.SemaphoreType.DMA,
        ],
    )
    def k(table_hbm, idx_hbm, out_hbm, idx_v, rows_v, sem):
        wid = lax.axis_index("s") * info.num_cores + lax.axis_index("c")
        pltpu.sync_copy(idx_hbm.at[pl.ds(wid * c_per_w, c_per_w)], idx_v)
        copies = [
            pltpu.async_copy(
                table_hbm.at[idx_v.at[j]],
                rows_v.at[pl.ds(j * chunk, chunk)], sem)
            for j in range(c_per_w)
        ]
        for c in copies:
            c.wait()
        pltpu.sync_copy(rows_v, out_hbm.at[pl.ds(wid * b_per_w, b_per_w)])

    return k(table, idx)


def _quantize_b_body(b_ref, rs_ref, cbt_ref, cb_ref, c2_ref, out_ref):
    g_all = b_ref[...] / rs_ref[...]
    cbt, cb, c2 = cbt_ref[...], cb_ref[...], c2_ref[...]
    rows = g_all.shape[0]
    deqs = []
    for j in range(128 // D_GRP):
        g = g_all[:, D_GRP * j:D_GRP * (j + 1)]
        oh = _pq_onehot(g, cbt, c2, rows)
        deqs.append(_dequant(oh, cb))
    out_ref[...] = jnp.concatenate(deqs, axis=1) * rs_ref[...]


def _quantize_b(b2d, rs_rep, cbt_m2, cb, c2):
    n, w = b2d.shape
    return pl.pallas_call(
        _quantize_b_body,
        out_shape=jax.ShapeDtypeStruct((n, w), jnp.float32),
    )(b2d, rs_rep, cbt_m2, cb, c2)


def _matmul_body(k, a_ref, b_ref, out_ref):
    out_ref[...] = jax.lax.dot_general(
        a_ref[...][:, :k], b_ref[...], (((1,), (0,)), ((), ())),
        preferred_element_type=jnp.float32,
    )


def _matmul(a, b):
    m, k_pad = a.shape
    k, n = b.shape
    return pl.pallas_call(
        functools.partial(_matmul_body, k),
        grid=(m // M_BLK,),
        in_specs=[
            pl.BlockSpec((M_BLK, k_pad), lambda i: (i, 0)),
            pl.BlockSpec((k, n), lambda i: (0, 0)),
        ],
        out_specs=pl.BlockSpec((M_BLK, n), lambda i: (i, 0)),
        out_shape=jax.ShapeDtypeStruct((m, n), jnp.float32),
    )(a, b)


def kernel(local_ids, U, B, rs_U, rs_B, codebook):
    o_u, i_u = U.shape
    o_b, i_b = B.shape
    cbt_m2 = codebook.T * -2.0
    c2 = (codebook * codebook).sum(-1)[None, :]

    table = _quantize_u(U, rs_U, cbt_m2, codebook, c2)

    ids2d = local_ids.astype(jnp.int32).reshape(-1, 128)
    uq_rows = _sc_gather(table, ids2d)

    b2d = B.reshape(-1, 128)
    rs_rep = jnp.repeat(rs_B, i_b // 128, axis=0)
    bq = _quantize_b(b2d, rs_rep, cbt_m2, codebook, c2).reshape(o_b, i_b)

    return _matmul(uq_rows, bq)
